# baseline (speedup 1.0000x reference)
import functools

import jax
import jax.numpy as jnp
from jax import lax
from jax.experimental import pallas as pl
from jax.experimental.pallas import tpu as pltpu
from jax.experimental.pallas import tpu_sc as plsc

NC = 2
NS = 16
NW = NC * NS

HT = 25
BT = 32
N_BLOCKS = HT * BT
BPW = N_BLOCKS // NW
PITCH = 32


@jax.jit
def _emb_lookup(idx4, table):
    mesh = plsc.VectorSubcoreMesh(core_axis_name="c", subcore_axis_name="s")

    @functools.partial(
        pl.kernel,
        out_type=jax.ShapeDtypeStruct((200, 4, 32, 8, 128), jnp.float32),
        mesh=mesh,
        scratch_types=[
            pltpu.VMEM((BPW, 8, 128), jnp.int32),
            pltpu.VMEM((1024, PITCH), jnp.float32),
            pltpu.VMEM((1024, PITCH), jnp.float32),
            pltpu.VMEM((8, 4, 8, 128), jnp.float32),
            pltpu.SemaphoreType.DMA,
            pltpu.SemaphoreType.DMA,
            pltpu.SemaphoreType.DMA,
        ],
        compiler_params=pltpu.CompilerParams(
            use_tc_tiling_on_sc=False, needs_layout_passes=False
        ),
    )
    def body(idx_hbm, table_hbm, out_hbm, idx_v, rows_v0, rows_v1,
             ot_v, gsem0, gsem1, osem):
        wid = lax.axis_index("s") * NC + lax.axis_index("c")
        blk0 = wid * BPW
        lane = lax.iota(jnp.int32, 16)
        pltpu.sync_copy(idx_hbm.at[pl.ds(blk0, BPW)], idx_v)

        def fire(blk, rows_v, gsem):
            k = blk - blk0
            for hr in range(8):
                pltpu.async_copy(
                    table_hbm.at[idx_v.at[k, hr]],
                    rows_v.at[pl.ds(hr * 128, 128)],
                    gsem,
                )

        def wait_gathers(blk, rows_v, gsem):
            k = blk - blk0
            for hr in range(8):
                pltpu.make_async_copy(
                    table_hbm.at[idx_v.at[k, hr]],
                    rows_v.at[pl.ds(hr * 128, 128)],
                    gsem,
                ).wait()

        def drain_writes():
            for _ in range(32):
                pltpu.make_async_copy(
                    ot_v.at[0, 0], out_hbm.at[0, 0, 0], osem
                ).wait()

        def transpose_and_write(blk, rows_v, gsem, first):
            ih = blk // BT
            jb = blk % BT
            wait_gathers(blk, rows_v, gsem)
            if not first:
                drain_writes()

            @plsc.parallel_loop(0, 32, 1, unroll=4)
            def tp_body(t):
                hr = t // 4
                i = t % 4
                base = hr * 128
                rvecs = [base + s * 16 + lane for s in range(8)]
                for r in range(8):
                    cvec = jnp.full((16,), 8 * i + r, jnp.int32)
                    for s in range(8):
                        vals = plsc.load_gather(rows_v, [rvecs[s], cvec])
                        ot_v[hr, i, r, pl.ds(s * 16, 16)] = vals
            for hr in range(8):
                for i in range(4):
                    pltpu.async_copy(
                        ot_v.at[hr, i], out_hbm.at[ih * 8 + hr, i, jb], osem
                    )

        fire(blk0, rows_v0, gsem0)

        def pair(j, carry):
            b0 = blk0 + 2 * j
            fire(b0 + 1, rows_v1, gsem1)
            transpose_and_write(b0, rows_v0, gsem0, False)
            fire(b0 + 2, rows_v0, gsem0)
            transpose_and_write(b0 + 1, rows_v1, gsem1, False)
            return carry

        fire(blk0 + 1, rows_v1, gsem1)
        transpose_and_write(blk0, rows_v0, gsem0, True)
        fire(blk0 + 2, rows_v0, gsem0)
        transpose_and_write(blk0 + 1, rows_v1, gsem1, False)
        lax.fori_loop(1, 12, pair, 0)
        transpose_and_write(blk0 + 24, rows_v0, gsem0, False)
        drain_writes()

    return body(idx4, table)


def kernel(inputs, emb_table):
    batch, hist = inputs.shape
    emb_dim = emb_table.shape[1]
    idx = inputs.astype(jnp.int32)
    idx4 = idx.reshape(BT, 128, HT, 8).transpose(2, 0, 3, 1).reshape(
        N_BLOCKS, 8, 128)
    out5 = _emb_lookup(idx4, emb_table)
    out = out5.transpose(2, 4, 0, 1, 3).reshape(batch, hist, emb_dim)
    return out


# speedup vs baseline: 1.1474x; 1.1474x over previous
import functools

import jax
import jax.numpy as jnp
from jax import lax
from jax.experimental import pallas as pl
from jax.experimental.pallas import tpu as pltpu
from jax.experimental.pallas import tpu_sc as plsc

NC = 2
NS = 16
NW = NC * NS

HT = 25
BT = 32
N_BLOCKS = HT * BT
BPW = N_BLOCKS // NW
PITCH = 32


@jax.jit
def _emb_lookup(idx4, table):
    mesh = plsc.VectorSubcoreMesh(core_axis_name="c", subcore_axis_name="s")

    @functools.partial(
        pl.kernel,
        out_type=jax.ShapeDtypeStruct((200, 4, 32, 8, 128), jnp.float32),
        mesh=mesh,
        scratch_types=[
            pltpu.VMEM((BPW, 8, 128), jnp.int32),
            pltpu.VMEM((1024, PITCH), jnp.float32),
            pltpu.VMEM((1024, PITCH), jnp.float32),
            pltpu.VMEM((8, 4, 8, 128), jnp.float32),
            pltpu.SemaphoreType.DMA,
            pltpu.SemaphoreType.DMA,
            pltpu.SemaphoreType.DMA,
        ],
        compiler_params=pltpu.CompilerParams(
            use_tc_tiling_on_sc=False, needs_layout_passes=False
        ),
    )
    def body(idx_hbm, table_hbm, out_hbm, idx_v, rows_v0, rows_v1,
             ot_v, gsem0, gsem1, osem):
        wid = lax.axis_index("s") * NC + lax.axis_index("c")
        blk0 = wid * BPW
        lane = lax.iota(jnp.int32, 16)
        pltpu.sync_copy(idx_hbm.at[pl.ds(blk0, BPW)], idx_v)

        def fire(blk, rows_v, gsem):
            k = blk - blk0
            for hr in range(8):
                pltpu.async_copy(
                    table_hbm.at[idx_v.at[k, hr]],
                    rows_v.at[pl.ds(hr * 128, 128)],
                    gsem,
                )

        def wait_gathers(blk, rows_v, gsem):
            k = blk - blk0
            for hr in range(8):
                pltpu.make_async_copy(
                    table_hbm.at[idx_v.at[k, hr]],
                    rows_v.at[pl.ds(hr * 128, 128)],
                    gsem,
                ).wait()

        def drain_writes():
            for _ in range(32):
                pltpu.make_async_copy(
                    ot_v.at[0, 0], out_hbm.at[0, 0, 0], osem
                ).wait()

        def transpose_and_write(blk, rows_v, gsem, first):
            ih = blk // BT
            jb = blk % BT
            wait_gathers(blk, rows_v, gsem)
            if not first:
                drain_writes()

            @plsc.parallel_loop(0, 32, 1, unroll=4)
            def tp_body(t):
                hr = t // 4
                i = t % 4
                base = hr * 128
                rvecs = [base + s * 16 + lane for s in range(8)]
                for r in range(8):
                    cvec = jnp.full((16,), 8 * i + r, jnp.int32)
                    for s in range(8):
                        vals = plsc.load_gather(rows_v, [rvecs[s], cvec])
                        ot_v[hr, i, r, pl.ds(s * 16, 16)] = vals
            for hr in range(8):
                for i in range(4):
                    pltpu.async_copy(
                        ot_v.at[hr, i], out_hbm.at[ih * 8 + hr, i, jb], osem
                    )

        fire(blk0, rows_v0, gsem0)

        def pair(j, carry):
            b0 = blk0 + 2 * j
            fire(b0 + 1, rows_v1, gsem1)
            transpose_and_write(b0, rows_v0, gsem0, False)
            fire(b0 + 2, rows_v0, gsem0)
            transpose_and_write(b0 + 1, rows_v1, gsem1, False)
            return carry

        fire(blk0 + 1, rows_v1, gsem1)
        transpose_and_write(blk0, rows_v0, gsem0, True)
        fire(blk0 + 2, rows_v0, gsem0)
        transpose_and_write(blk0 + 1, rows_v1, gsem1, False)
        lax.fori_loop(1, 12, pair, 0)
        transpose_and_write(blk0 + 24, rows_v0, gsem0, False)
        drain_writes()

    return body(idx4, table)


def _detile_body(x_ref, o_ref):
    xt = x_ref[...].T
    xt3 = xt.reshape(o_ref.shape[0], 4, 32)
    for t in range(4):
        o_ref[:, 32 * t:32 * (t + 1)] = xt3[:, t, :]


def _detile_table(table_t):
    vb = 4096
    grid = (table_t.shape[1] + vb - 1) // vb
    return pl.pallas_call(
        _detile_body,
        grid=(grid,),
        in_specs=[pl.BlockSpec((32, vb), lambda i: (0, i))],
        out_specs=pl.BlockSpec((vb // 4, 128), lambda i: (i, 0)),
        out_shape=jax.ShapeDtypeStruct((250000, 128), jnp.float32),
    )(table_t)


def kernel(inputs, emb_table):
    batch, hist = inputs.shape
    emb_dim = emb_table.shape[1]
    idx = inputs.astype(jnp.int32)
    idx4 = idx.reshape(BT, 128, HT, 8).transpose(2, 0, 3, 1).reshape(
        N_BLOCKS, 8, 128)
    tlin = _detile_table(emb_table.T).reshape(-1).reshape(
        emb_table.shape[0], emb_dim)
    out5 = _emb_lookup(idx4, tlin)
    out = out5.transpose(2, 4, 0, 1, 3).reshape(batch, hist, emb_dim)
    return out


# speedup vs baseline: 1.1832x; 1.0313x over previous
import functools

import jax
import jax.numpy as jnp
from jax import lax
from jax.experimental import pallas as pl
from jax.experimental.pallas import tpu as pltpu
from jax.experimental.pallas import tpu_sc as plsc

NC = 2
NS = 16
NW = NC * NS

HT = 25
BT = 32
N_BLOCKS = HT * BT
BPW = N_BLOCKS // NW
PITCH = 32


@jax.jit
def _emb_lookup(idx4, table):
    mesh = plsc.VectorSubcoreMesh(core_axis_name="c", subcore_axis_name="s")

    @functools.partial(
        pl.kernel,
        out_type=jax.ShapeDtypeStruct((200, 4, 32, 8, 128), jnp.float32),
        mesh=mesh,
        scratch_types=[
            pltpu.VMEM((BPW, 8, 128), jnp.int32),
            pltpu.VMEM((1024, PITCH), jnp.float32),
            pltpu.VMEM((1024, PITCH), jnp.float32),
            pltpu.VMEM((8, 4, 8, 128), jnp.float32),
            pltpu.SemaphoreType.DMA,
            pltpu.SemaphoreType.DMA,
            pltpu.SemaphoreType.DMA,
        ],
        compiler_params=pltpu.CompilerParams(
            use_tc_tiling_on_sc=False, needs_layout_passes=False
        ),
    )
    def body(idx_hbm, table_hbm, out_hbm, idx_v, rows_v0, rows_v1,
             ot_v, gsem0, gsem1, osem):
        wid = lax.axis_index("s") * NC + lax.axis_index("c")
        blk0 = wid * BPW
        lane = lax.iota(jnp.int32, 16)
        pltpu.sync_copy(idx_hbm.at[pl.ds(blk0, BPW)], idx_v)

        def fire(blk, rows_v, gsem):
            k = blk - blk0
            for hr in range(8):
                pltpu.async_copy(
                    table_hbm.at[idx_v.at[k, hr]],
                    rows_v.at[pl.ds(hr * 128, 128)],
                    gsem,
                )

        def wait_gathers(blk, rows_v, gsem):
            k = blk - blk0
            for hr in range(8):
                pltpu.make_async_copy(
                    table_hbm.at[idx_v.at[k, hr]],
                    rows_v.at[pl.ds(hr * 128, 128)],
                    gsem,
                ).wait()

        def drain_writes():
            for _ in range(8):
                pltpu.make_async_copy(
                    ot_v.at[0], out_hbm.at[0, :, 0], osem
                ).wait()

        def transpose_and_write(blk, rows_v, gsem, first):
            ih = blk // BT
            jb = blk % BT
            wait_gathers(blk, rows_v, gsem)
            if not first:
                drain_writes()

            @plsc.parallel_loop(0, 32, 1, unroll=4)
            def tp_body(t):
                hr = t // 4
                i = t % 4
                base = hr * 128
                rvecs = [base + s * 16 + lane for s in range(8)]
                for r in range(8):
                    cvec = jnp.full((16,), 8 * i + r, jnp.int32)
                    for s in range(8):
                        vals = plsc.load_gather(rows_v, [rvecs[s], cvec])
                        ot_v[hr, i, r, pl.ds(s * 16, 16)] = vals
            for hr in range(8):
                pltpu.async_copy(
                    ot_v.at[hr], out_hbm.at[ih * 8 + hr, :, jb], osem
                )

        fire(blk0, rows_v0, gsem0)

        def pair(j, carry):
            b0 = blk0 + 2 * j
            fire(b0 + 1, rows_v1, gsem1)
            transpose_and_write(b0, rows_v0, gsem0, False)
            fire(b0 + 2, rows_v0, gsem0)
            transpose_and_write(b0 + 1, rows_v1, gsem1, False)
            return carry

        fire(blk0 + 1, rows_v1, gsem1)
        transpose_and_write(blk0, rows_v0, gsem0, True)
        fire(blk0 + 2, rows_v0, gsem0)
        transpose_and_write(blk0 + 1, rows_v1, gsem1, False)
        lax.fori_loop(1, 12, pair, 0)
        transpose_and_write(blk0 + 24, rows_v0, gsem0, False)
        drain_writes()

    return body(idx4, table)


def _detile_body(x_ref, o_ref):
    xt = x_ref[...].T
    xt3 = xt.reshape(o_ref.shape[0], 4, 32)
    for t in range(4):
        o_ref[:, 32 * t:32 * (t + 1)] = xt3[:, t, :]


def _detile_table(table_t):
    vb = 8192
    grid = (table_t.shape[1] + vb - 1) // vb
    return pl.pallas_call(
        _detile_body,
        grid=(grid,),
        in_specs=[pl.BlockSpec((32, vb), lambda i: (0, i))],
        out_specs=pl.BlockSpec((vb // 4, 128), lambda i: (i, 0)),
        out_shape=jax.ShapeDtypeStruct((250000, 128), jnp.float32),
    )(table_t)


def kernel(inputs, emb_table):
    batch, hist = inputs.shape
    emb_dim = emb_table.shape[1]
    idx = inputs.astype(jnp.int32)
    idx4 = idx.reshape(BT, 128, HT, 8).transpose(2, 0, 3, 1).reshape(
        N_BLOCKS, 8, 128)
    tlin = _detile_table(emb_table.T).reshape(-1).reshape(
        emb_table.shape[0], emb_dim)
    out5 = _emb_lookup(idx4, tlin)
    out = out5.transpose(2, 4, 0, 1, 3).reshape(batch, hist, emb_dim)
    return out
